# Initial kernel scaffold; baseline (speedup 1.0000x reference)
import jax
import jax.numpy as jnp
from jax.experimental import pallas as pl


def kernel(queries, candidates, identifiers, k):
    raise NotImplementedError("write your pallas kernel here")


# speedup vs baseline: 2.0345x; 2.0345x over previous
import jax
import jax.numpy as jnp
from jax.experimental import pallas as pl
from jax.experimental.pallas import tpu as pltpu

Q = 1024
D = 128
N = 100000
K = 10
QB = 256
CB = 2000
NQB = Q // QB
NCB = N // CB
RUNW = 16
NEG = float("-inf")
BIGI = jnp.iinfo(jnp.int32).max


def _topk_body(q_ref, c_ref, os_ref, oi_ref, run_s, run_i):
    b = pl.program_id(1)

    @pl.when(b == 0)
    def _init():
        run_s[...] = jnp.full((QB, RUNW), NEG, jnp.float32)
        run_i[...] = jnp.full((QB, RUNW), BIGI, jnp.int32)

    s = jax.lax.dot_general(
        q_ref[...], c_ref[...], (((1,), (1,)), ((), ())),
        preferred_element_type=jnp.float32)
    ids = jax.lax.broadcasted_iota(jnp.int32, (QB, CB), 1) + b * CB

    x = jnp.concatenate([s, run_s[...]], axis=1)
    xi = jnp.concatenate([ids, run_i[...]], axis=1)

    top_s, top_i = [], []
    for _ in range(K):
        m = jnp.max(x, axis=1, keepdims=True)
        aid = jnp.min(jnp.where(x == m, xi, BIGI), axis=1, keepdims=True)
        top_s.append(m)
        top_i.append(aid)
        x = jnp.where(xi == aid, NEG, x)
    ts = jnp.concatenate(top_s, axis=1)
    ti = jnp.concatenate(top_i, axis=1)

    run_s[...] = jnp.concatenate(
        [ts, jnp.full((QB, RUNW - K), NEG, jnp.float32)], axis=1)
    run_i[...] = jnp.concatenate(
        [ti, jnp.full((QB, RUNW - K), BIGI, jnp.int32)], axis=1)

    @pl.when(b == NCB - 1)
    def _out():
        os_ref[...] = ts
        oi_ref[...] = ti


def kernel(queries, candidates, identifiers, k):
    assert queries.shape == (Q, D) and candidates.shape == (N, D)
    ts, ti = pl.pallas_call(
        _topk_body,
        grid=(NQB, NCB),
        in_specs=[
            pl.BlockSpec((QB, D), lambda qb, b: (qb, 0)),
            pl.BlockSpec((CB, D), lambda qb, b: (b, 0)),
        ],
        out_specs=[
            pl.BlockSpec((QB, K), lambda qb, b: (qb, 0)),
            pl.BlockSpec((QB, K), lambda qb, b: (qb, 0)),
        ],
        out_shape=[
            jax.ShapeDtypeStruct((Q, K), jnp.float32),
            jax.ShapeDtypeStruct((Q, K), jnp.int32),
        ],
        scratch_shapes=[
            pltpu.VMEM((QB, RUNW), jnp.float32),
            pltpu.VMEM((QB, RUNW), jnp.int32),
        ],
        compiler_params=pltpu.CompilerParams(
            dimension_semantics=("arbitrary", "arbitrary")),
    )(queries, candidates)
    top_ids = jnp.take(identifiers, ti, axis=0)
    return ts, top_ids
